# Initial kernel scaffold; baseline (speedup 1.0000x reference)
import jax
import jax.numpy as jnp
from jax.experimental import pallas as pl


def kernel(x, edge_index, W1, b1, g1, bt1, W2, b2, g2, bt2, Wd1, bd1, Wd2, bd2):
    raise NotImplementedError("write your pallas kernel here")


# speedup vs baseline: 2.4794x; 2.4794x over previous
import functools
import jax
import jax.numpy as jnp
from jax.experimental import pallas as pl
from jax.experimental.pallas import tpu as pltpu

N = 10000
RB = 400
GRID = N // RB


def _k1_body(x_ref, w_ref, deg_ref, z_ref, dinv_ref):
    i = pl.program_id(0)
    dinv = jax.lax.rsqrt(deg_ref[pl.ds(i * RB, RB), :])
    z_ref[...] = jnp.dot(x_ref[...], w_ref[...],
                         preferred_element_type=jnp.float32) * dinv
    dinv_ref[...] = jax.lax.rsqrt(deg_ref[...])


def _mm_scale(x, W, deg2d):
    D_in, D_out = W.shape
    return pl.pallas_call(
        _k1_body,
        grid=(GRID,),
        in_specs=[
            pl.BlockSpec((RB, D_in), lambda i: (i, 0)),
            pl.BlockSpec((D_in, D_out), lambda i: (0, 0)),
            pl.BlockSpec((N, 1), lambda i: (0, 0)),
        ],
        out_specs=[
            pl.BlockSpec((RB, D_out), lambda i: (i, 0)),
            pl.BlockSpec((N, 1), lambda i: (0, 0)),
        ],
        out_shape=[
            jax.ShapeDtypeStruct((N, D_out), jnp.float32),
            jax.ShapeDtypeStruct((N, 1), jnp.float32),
        ],
    )(x, W, deg2d)


def _kA_body(s_ref, z_ref, dinv_ref, b_ref, t_ref, st_ref):
    i = pl.program_id(0)
    dinv = dinv_ref[pl.ds(i * RB, RB), :]
    t = dinv * (s_ref[...] + z_ref[...]) + b_ref[...]
    t_ref[...] = t
    colsum = jnp.sum(t, axis=0, keepdims=True)
    colsq = jnp.sum(t * t, axis=0, keepdims=True)
    upd = jnp.concatenate([colsum, colsq, jnp.zeros((6, t.shape[1]),
                                                    jnp.float32)], axis=0)

    @pl.when(i == 0)
    def _():
        st_ref[...] = jnp.zeros_like(st_ref)

    st_ref[...] += upd


def _post_conv(S, z, dinv2d, b):
    D = z.shape[1]
    return pl.pallas_call(
        _kA_body,
        grid=(GRID,),
        in_specs=[
            pl.BlockSpec((RB, D), lambda i: (i, 0)),
            pl.BlockSpec((RB, D), lambda i: (i, 0)),
            pl.BlockSpec((N, 1), lambda i: (0, 0)),
            pl.BlockSpec((1, D), lambda i: (0, 0)),
        ],
        out_specs=[
            pl.BlockSpec((RB, D), lambda i: (i, 0)),
            pl.BlockSpec((8, D), lambda i: (0, 0)),
        ],
        out_shape=[
            jax.ShapeDtypeStruct((N, D), jnp.float32),
            jax.ShapeDtypeStruct((8, D), jnp.float32),
        ],
    )(S, z, dinv2d, b)


def _lrelu(x):
    return jnp.where(x >= 0, x, 0.01 * x)


def _bn_from_stats(t, st_ref, g_ref, bt_ref):
    mean = st_ref[0:1, :] * (1.0 / N)
    var = st_ref[1:2, :] * (1.0 / N) - mean * mean
    return (t - mean) * jax.lax.rsqrt(var + 1e-5) * g_ref[...] + bt_ref[...]


def _kB_body(t_ref, st_ref, g_ref, bt_ref, w_ref, dinv_ref, z_ref):
    i = pl.program_id(0)
    y = _lrelu(_bn_from_stats(t_ref[...], st_ref, g_ref, bt_ref))
    dinv = dinv_ref[pl.ds(i * RB, RB), :]
    z_ref[...] = jnp.dot(y, w_ref[...],
                         preferred_element_type=jnp.float32) * dinv


def _bn_mm_scale(t, st, g, bt, W, dinv2d):
    D_in, D_out = W.shape
    return pl.pallas_call(
        _kB_body,
        grid=(GRID,),
        in_specs=[
            pl.BlockSpec((RB, D_in), lambda i: (i, 0)),
            pl.BlockSpec((8, D_in), lambda i: (0, 0)),
            pl.BlockSpec((1, D_in), lambda i: (0, 0)),
            pl.BlockSpec((1, D_in), lambda i: (0, 0)),
            pl.BlockSpec((D_in, D_out), lambda i: (0, 0)),
            pl.BlockSpec((N, 1), lambda i: (0, 0)),
        ],
        out_specs=pl.BlockSpec((RB, D_out), lambda i: (i, 0)),
        out_shape=jax.ShapeDtypeStruct((N, D_out), jnp.float32),
    )(t, st, g.reshape(1, -1), bt.reshape(1, -1), W, dinv2d)


def _kC_body(t_ref, st_ref, g_ref, bt_ref, w1_ref, b1_ref, w2_ref, b2_ref,
             o_ref):
    y = _lrelu(_bn_from_stats(t_ref[...], st_ref, g_ref, bt_ref))
    h = _lrelu(jnp.dot(y, w1_ref[...], preferred_element_type=jnp.float32)
               + b1_ref[...])
    logits = jnp.dot(h, w2_ref[...], preferred_element_type=jnp.float32) \
        + b2_ref[...]
    col = jax.lax.broadcasted_iota(jnp.int32, logits.shape, 1)
    logits = jnp.where(col < 6, logits, -1e30)
    m = jnp.max(logits, axis=1, keepdims=True)
    e = jnp.exp(logits - m)
    o_ref[...] = e / jnp.sum(e, axis=1, keepdims=True)


def _head(t, st, g, bt, Wd1, bd1, Wd2p, bd2p):
    D = t.shape[1]
    H = Wd1.shape[1]
    return pl.pallas_call(
        _kC_body,
        grid=(GRID,),
        in_specs=[
            pl.BlockSpec((RB, D), lambda i: (i, 0)),
            pl.BlockSpec((8, D), lambda i: (0, 0)),
            pl.BlockSpec((1, D), lambda i: (0, 0)),
            pl.BlockSpec((1, D), lambda i: (0, 0)),
            pl.BlockSpec((D, H), lambda i: (0, 0)),
            pl.BlockSpec((1, H), lambda i: (0, 0)),
            pl.BlockSpec((H, 128), lambda i: (0, 0)),
            pl.BlockSpec((1, 128), lambda i: (0, 0)),
        ],
        out_specs=pl.BlockSpec((RB, 128), lambda i: (i, 0)),
        out_shape=jax.ShapeDtypeStruct((N, 128), jnp.float32),
    )(t, st, g.reshape(1, -1), bt.reshape(1, -1), Wd1, bd1.reshape(1, -1),
      Wd2p, bd2p.reshape(1, -1))


def _deg(edge_index):
    dst = edge_index[1]
    deg = jax.ops.segment_sum(jnp.ones(dst.shape[0], jnp.float32), dst,
                              num_segments=N)
    return (deg + 1.0).reshape(N, 1)


def _segsum(z, edge_index):
    src = edge_index[0]
    dst = edge_index[1]
    return jax.ops.segment_sum(z[src], dst, num_segments=N)


def kernel(x, edge_index, W1, b1, g1, bt1, W2, b2, g2, bt2, Wd1, bd1, Wd2,
           bd2):
    deg2d = _deg(edge_index)
    z1, dinv2d = _mm_scale(x, W1, deg2d)
    S1 = _segsum(z1, edge_index)
    t1, st1 = _post_conv(S1, z1, dinv2d, b1.reshape(1, -1))
    z2 = _bn_mm_scale(t1, st1, g1, bt1, W2, dinv2d)
    S2 = _segsum(z2, edge_index)
    t2, st2 = _post_conv(S2, z2, dinv2d, b2.reshape(1, -1))
    Wd2p = jnp.pad(Wd2, ((0, 0), (0, 128 - Wd2.shape[1])))
    bd2p = jnp.pad(bd2, (0, 128 - bd2.shape[0]))
    out = _head(t2, st2, g2, bt2, Wd1, bd1, Wd2p, bd2p)
    return out[:, :6]


# speedup vs baseline: 5.7842x; 2.3329x over previous
import functools
import jax
import jax.numpy as jnp
from jax import lax
from jax.experimental import pallas as pl
from jax.experimental.pallas import tpu as pltpu
from jax.experimental.pallas import tpu_sc as plsc

N = 10000
RB = 400
GRID = N // RB

NC, NS, LANES = 2, 16, 16
E = 320000
IDXW = 128
EROWS = 2560
RPT = EROWS // NS
STG = 8
ACC_N = 10240
DUMP = 10100
RPS = ACC_N // NS


@functools.lru_cache(maxsize=None)
def _mesh():
    return plsc.VectorSubcoreMesh(core_axis_name="c", subcore_axis_name="s",
                                  num_cores=NC, num_subcores=NS)


def _k1_body(x_ref, w_ref, deg_ref, *out_refs):
    i = pl.program_id(0)
    z_refs, dinv_ref = out_refs[:-1], out_refs[-1]
    dinv = jax.lax.rsqrt(deg_ref[pl.ds(i * RB, RB), :] + 1.0)
    zz = jnp.dot(x_ref[...], w_ref[...],
                 preferred_element_type=jnp.float32) * dinv
    for k, zr in enumerate(z_refs):
        zr[...] = zz[:, k * 128:(k + 1) * 128]
    dinv_ref[...] = jax.lax.rsqrt(deg_ref[...] + 1.0)


def _mm_scale(x, W, deg2d):
    D_in, D_out = W.shape
    nch = D_out // 128
    out = pl.pallas_call(
        _k1_body,
        grid=(GRID,),
        in_specs=[
            pl.BlockSpec((RB, D_in), lambda i: (i, 0)),
            pl.BlockSpec((D_in, D_out), lambda i: (0, 0)),
            pl.BlockSpec((N, 1), lambda i: (0, 0)),
        ],
        out_specs=[pl.BlockSpec((RB, 128), lambda i: (i, 0))] * nch
        + [pl.BlockSpec((N, 1), lambda i: (0, 0))],
        out_shape=[jax.ShapeDtypeStruct((N, 128), jnp.float32)] * nch
        + [jax.ShapeDtypeStruct((N, 1), jnp.float32)],
    )(x, W, deg2d)
    return out[:-1], out[-1]


def _kA_body(*refs):
    nch = (len(refs) - 4) // 2
    s_refs = refs[:nch]
    z_refs = refs[nch:2 * nch]
    dinv_ref, b_ref, t_ref, st_ref = refs[2 * nch:]
    i = pl.program_id(0)
    dinv = dinv_ref[pl.ds(i * RB, RB), :]
    S = jnp.concatenate([r[...] for r in s_refs], axis=1)
    z = jnp.concatenate([r[...] for r in z_refs], axis=1)
    t = dinv * (S + z) + b_ref[...]
    t_ref[...] = t
    colsum = jnp.sum(t, axis=0, keepdims=True)
    colsq = jnp.sum(t * t, axis=0, keepdims=True)
    upd = jnp.concatenate([colsum, colsq, jnp.zeros((6, t.shape[1]),
                                                    jnp.float32)], axis=0)

    @pl.when(i == 0)
    def _():
        st_ref[...] = jnp.zeros_like(st_ref)

    st_ref[...] += upd


def _post_conv(S_chunks, z_chunks, dinv2d, b):
    nch = len(z_chunks)
    D = nch * 128
    return pl.pallas_call(
        _kA_body,
        grid=(GRID,),
        in_specs=[pl.BlockSpec((RB, 128), lambda i: (i, 0))] * (2 * nch)
        + [
            pl.BlockSpec((N, 1), lambda i: (0, 0)),
            pl.BlockSpec((1, D), lambda i: (0, 0)),
        ],
        out_specs=[
            pl.BlockSpec((RB, D), lambda i: (i, 0)),
            pl.BlockSpec((8, D), lambda i: (0, 0)),
        ],
        out_shape=[
            jax.ShapeDtypeStruct((N, D), jnp.float32),
            jax.ShapeDtypeStruct((8, D), jnp.float32),
        ],
    )(*S_chunks, *z_chunks, dinv2d, b)


def _lrelu(x):
    return jnp.where(x >= 0, x, 0.01 * x)


def _bn_from_stats(t, st_ref, g_ref, bt_ref):
    mean = st_ref[0:1, :] * (1.0 / N)
    var = st_ref[1:2, :] * (1.0 / N) - mean * mean
    return (t - mean) * jax.lax.rsqrt(var + 1e-5) * g_ref[...] + bt_ref[...]


def _kB_body(t_ref, st_ref, g_ref, bt_ref, w_ref, dinv_ref, *z_refs):
    i = pl.program_id(0)
    y = _lrelu(_bn_from_stats(t_ref[...], st_ref, g_ref, bt_ref))
    dinv = dinv_ref[pl.ds(i * RB, RB), :]
    zz = jnp.dot(y, w_ref[...], preferred_element_type=jnp.float32) * dinv
    for k, zr in enumerate(z_refs):
        zr[...] = zz[:, k * 128:(k + 1) * 128]


def _bn_mm_scale(t, st, g, bt, W, dinv2d):
    D_in, D_out = W.shape
    nch = D_out // 128
    return pl.pallas_call(
        _kB_body,
        grid=(GRID,),
        in_specs=[
            pl.BlockSpec((RB, D_in), lambda i: (i, 0)),
            pl.BlockSpec((8, D_in), lambda i: (0, 0)),
            pl.BlockSpec((1, D_in), lambda i: (0, 0)),
            pl.BlockSpec((1, D_in), lambda i: (0, 0)),
            pl.BlockSpec((D_in, D_out), lambda i: (0, 0)),
            pl.BlockSpec((N, 1), lambda i: (0, 0)),
        ],
        out_specs=[pl.BlockSpec((RB, 128), lambda i: (i, 0))] * nch,
        out_shape=[jax.ShapeDtypeStruct((N, 128), jnp.float32)] * nch,
    )(t, st, g.reshape(1, -1), bt.reshape(1, -1), W, dinv2d)


def _kC_body(t_ref, st_ref, g_ref, bt_ref, w1_ref, b1_ref, w2_ref, b2_ref,
             o_ref):
    y = _lrelu(_bn_from_stats(t_ref[...], st_ref, g_ref, bt_ref))
    h = _lrelu(jnp.dot(y, w1_ref[...], preferred_element_type=jnp.float32)
               + b1_ref[...])
    logits = jnp.dot(h, w2_ref[...], preferred_element_type=jnp.float32) \
        + b2_ref[...]
    col = jax.lax.broadcasted_iota(jnp.int32, logits.shape, 1)
    logits = jnp.where(col < 6, logits, -1e30)
    m = jnp.max(logits, axis=1, keepdims=True)
    e = jnp.exp(logits - m)
    o_ref[...] = e / jnp.sum(e, axis=1, keepdims=True)


def _head(t, st, g, bt, Wd1, bd1, Wd2p, bd2p):
    D = t.shape[1]
    H = Wd1.shape[1]
    return pl.pallas_call(
        _kC_body,
        grid=(GRID,),
        in_specs=[
            pl.BlockSpec((RB, D), lambda i: (i, 0)),
            pl.BlockSpec((8, D), lambda i: (0, 0)),
            pl.BlockSpec((1, D), lambda i: (0, 0)),
            pl.BlockSpec((1, D), lambda i: (0, 0)),
            pl.BlockSpec((D, H), lambda i: (0, 0)),
            pl.BlockSpec((1, H), lambda i: (0, 0)),
            pl.BlockSpec((H, 128), lambda i: (0, 0)),
            pl.BlockSpec((1, 128), lambda i: (0, 0)),
        ],
        out_specs=pl.BlockSpec((RB, 128), lambda i: (i, 0)),
        out_shape=jax.ShapeDtypeStruct((N, 128), jnp.float32),
    )(t, st, g.reshape(1, -1), bt.reshape(1, -1), Wd1, bd1.reshape(1, -1),
      Wd2p, bd2p.reshape(1, -1))


HPAD = 10240
DEGR = EROWS // NS


@functools.lru_cache(maxsize=None)
def _make_degree():
    @functools.partial(
        pl.kernel,
        out_type=jax.ShapeDtypeStruct((HPAD, IDXW), jnp.float32),
        mesh=_mesh(),
        scratch_types=[
            pltpu.VMEM_SHARED((HPAD, IDXW), jnp.float32),
            pltpu.VMEM((16, IDXW), jnp.int32),
            pltpu.VMEM((IDXW, IDXW), jnp.float32),
            pltpu.VMEM((IDXW, IDXW), jnp.float32),
        ],
    )
    def degk(dst2d_hbm, deg_out, acc, idxv, onesb, zerob):
        c = lax.axis_index("c")
        s = lax.axis_index("s")

        def fill(i, _):
            for cc in range(IDXW // LANES):
                onesb[i, pl.ds(cc * LANES, LANES)] = jnp.ones((LANES,),
                                                              jnp.float32)
                zerob[i, pl.ds(cc * LANES, LANES)] = jnp.zeros((LANES,),
                                                               jnp.float32)
            return 0
        lax.fori_loop(0, IDXW, fill, 0)

        for q in range(HPAD // NS // IDXW):
            pltpu.sync_copy(zerob, acc.at[pl.ds(s * (HPAD // NS) + q * IDXW,
                                                IDXW)])
        plsc.subcore_barrier()

        @pl.when(c == 0)
        def _():
            def blk_body(blk, _):
                pltpu.sync_copy(
                    dst2d_hbm.at[pl.ds(s * DEGR + blk * 16, 16)], idxv)
                for j in range(16):
                    pltpu.sync_copy(onesb, acc.at[idxv.at[j]], add=True)
                return 0
            lax.fori_loop(0, DEGR // 16, blk_body, 0)
        plsc.subcore_barrier()

        @pl.when(c == 0)
        def _():
            for q in range(HPAD // NS // IDXW):
                pltpu.sync_copy(
                    acc.at[pl.ds(s * (HPAD // NS) + q * IDXW, IDXW)],
                    deg_out.at[pl.ds(s * (HPAD // NS) + q * IDXW, IDXW)])

    return degk


@functools.lru_cache(maxsize=None)
def _make_segsum(nchunks):
    npass = nchunks // NC

    @functools.partial(
        pl.kernel,
        out_type=[jax.ShapeDtypeStruct((ACC_N, IDXW), jnp.float32)] * nchunks,
        mesh=_mesh(),
        scratch_types=[
            pltpu.VMEM_SHARED((ACC_N, IDXW), jnp.float32),
            pltpu.VMEM((STG, IDXW), jnp.int32),
            pltpu.VMEM((STG, IDXW), jnp.int32),
            pltpu.VMEM((IDXW, IDXW), jnp.float32),
        ],
    )
    def seg(*args):
        zs = args[:nchunks]
        srcp, dstp = args[nchunks:nchunks + 2]
        outs = args[nchunks + 2:2 * nchunks + 2]
        acc, srcrows, dstrows, rowbuf = args[2 * nchunks + 2:]
        c = lax.axis_index("c")
        s = lax.axis_index("s")

        for p in range(npass):
            def zr(i, _):
                for cc in range(IDXW // LANES):
                    rowbuf[i, pl.ds(cc * LANES, LANES)] = \
                        jnp.zeros((LANES,), jnp.float32)
                return 0
            lax.fori_loop(0, IDXW, zr, 0)
            for q in range(RPS // IDXW):
                pltpu.sync_copy(rowbuf,
                                acc.at[pl.ds(s * RPS + q * IDXW, IDXW)])
            plsc.subcore_barrier()

            for cid in range(NC):
                @pl.when(c == cid)
                def _(cid=cid):
                    z_ref = zs[p * NC + cid]

                    def blk_body(blk, _):
                        base = s * RPT + blk * STG
                        pltpu.sync_copy(srcp.at[pl.ds(base, STG)], srcrows)
                        pltpu.sync_copy(dstp.at[pl.ds(base, STG)], dstrows)
                        for j in range(STG):
                            pltpu.sync_copy(z_ref.at[srcrows.at[j]], rowbuf)
                            pltpu.sync_copy(rowbuf, acc.at[dstrows.at[j]],
                                            add=True)
                        return 0
                    lax.fori_loop(0, RPT // STG, blk_body, 0)
            plsc.subcore_barrier()

            for cid in range(NC):
                @pl.when(c == cid)
                def _(cid=cid):
                    o_ref = outs[p * NC + cid]
                    for q in range(RPS // IDXW):
                        pltpu.sync_copy(
                            acc.at[pl.ds(s * RPS + q * IDXW, IDXW)],
                            o_ref.at[pl.ds(s * RPS + q * IDXW, IDXW)])
            if p + 1 < npass:
                plsc.subcore_barrier()

    return seg


def _edge_rows(edge_index):
    pad = EROWS * IDXW - E
    srcp = jnp.concatenate(
        [edge_index[0], jnp.zeros((pad,), jnp.int32)]).reshape(EROWS, IDXW)
    dstp = jnp.concatenate(
        [edge_index[1],
         jnp.full((pad,), DUMP, jnp.int32)]).reshape(EROWS, IDXW)
    return srcp, dstp


def _deg(dstp):
    return _make_degree()(dstp)[:N, 0:1]


def _segsum(z_chunks, srcp, dstp):
    return _make_segsum(len(z_chunks))(*z_chunks, srcp, dstp)


def kernel(x, edge_index, W1, b1, g1, bt1, W2, b2, g2, bt2, Wd1, bd1, Wd2,
           bd2):
    srcp, dstp = _edge_rows(edge_index)
    deg2d = _deg(dstp)
    z1c, dinv2d = _mm_scale(x, W1, deg2d)
    S1c = _segsum(z1c, srcp, dstp)
    t1, st1 = _post_conv(S1c, z1c, dinv2d, b1.reshape(1, -1))
    z2c = _bn_mm_scale(t1, st1, g1, bt1, W2, dinv2d)
    S2c = _segsum(z2c, srcp, dstp)
    t2, st2 = _post_conv(S2c, z2c, dinv2d, b2.reshape(1, -1))
    Wd2p = jnp.pad(Wd2, ((0, 0), (0, 128 - Wd2.shape[1])))
    bd2p = jnp.pad(bd2, (0, 128 - bd2.shape[0]))
    out = _head(t2, st2, g2, bt2, Wd1, bd1, Wd2p, bd2p)
    return out[:, :6]


# speedup vs baseline: 7.1302x; 1.2327x over previous
import functools
import jax
import jax.numpy as jnp
from jax import lax
from jax.experimental import pallas as pl
from jax.experimental.pallas import tpu as pltpu
from jax.experimental.pallas import tpu_sc as plsc

N = 10000
RB = 400
GRID = N // RB

NC, NS, LANES = 2, 16, 16
E = 320000
IDXW = 128
EROWS = 2560
RPT = EROWS // NS
STG = 8
ACC_N = 10240
DUMP = 10100
RPS = ACC_N // NS


@functools.lru_cache(maxsize=None)
def _mesh():
    return plsc.VectorSubcoreMesh(core_axis_name="c", subcore_axis_name="s",
                                  num_cores=NC, num_subcores=NS)


def _k1_body(x_ref, w_ref, deg_ref, *out_refs):
    i = pl.program_id(0)
    z_refs, dinv_ref = out_refs[:-1], out_refs[-1]
    dinv = jax.lax.rsqrt(deg_ref[pl.ds(i * RB, RB), :] + 1.0)
    zz = jnp.dot(x_ref[...], w_ref[...],
                 preferred_element_type=jnp.float32) * dinv
    for k, zr in enumerate(z_refs):
        zr[...] = zz[:, k * 128:(k + 1) * 128]
    dinv_ref[...] = jax.lax.rsqrt(deg_ref[...] + 1.0)


def _mm_scale(x, W, deg2d):
    D_in, D_out = W.shape
    nch = D_out // 128
    out = pl.pallas_call(
        _k1_body,
        grid=(GRID,),
        in_specs=[
            pl.BlockSpec((RB, D_in), lambda i: (i, 0)),
            pl.BlockSpec((D_in, D_out), lambda i: (0, 0)),
            pl.BlockSpec((N, 1), lambda i: (0, 0)),
        ],
        out_specs=[pl.BlockSpec((RB, 128), lambda i: (i, 0))] * nch
        + [pl.BlockSpec((N, 1), lambda i: (0, 0))],
        out_shape=[jax.ShapeDtypeStruct((N, 128), jnp.float32)] * nch
        + [jax.ShapeDtypeStruct((N, 1), jnp.float32)],
    )(x, W, deg2d)
    return out[:-1], out[-1]


def _kA_body(*refs):
    nch = (len(refs) - 4) // 2
    s_refs = refs[:nch]
    z_refs = refs[nch:2 * nch]
    dinv_ref, b_ref, t_ref, st_ref = refs[2 * nch:]
    i = pl.program_id(0)
    dinv = dinv_ref[pl.ds(i * RB, RB), :]
    S = jnp.concatenate([r[...] for r in s_refs], axis=1)
    z = jnp.concatenate([r[...] for r in z_refs], axis=1)
    t = dinv * (S + z) + b_ref[...]
    t_ref[...] = t
    colsum = jnp.sum(t, axis=0, keepdims=True)
    colsq = jnp.sum(t * t, axis=0, keepdims=True)
    upd = jnp.concatenate([colsum, colsq, jnp.zeros((6, t.shape[1]),
                                                    jnp.float32)], axis=0)

    @pl.when(i == 0)
    def _():
        st_ref[...] = jnp.zeros_like(st_ref)

    st_ref[...] += upd


def _post_conv(S_chunks, z_chunks, dinv2d, b):
    nch = len(z_chunks)
    D = nch * 128
    return pl.pallas_call(
        _kA_body,
        grid=(GRID,),
        in_specs=[pl.BlockSpec((RB, 128), lambda i: (i, 0))] * (2 * nch)
        + [
            pl.BlockSpec((N, 1), lambda i: (0, 0)),
            pl.BlockSpec((1, D), lambda i: (0, 0)),
        ],
        out_specs=[
            pl.BlockSpec((RB, D), lambda i: (i, 0)),
            pl.BlockSpec((8, D), lambda i: (0, 0)),
        ],
        out_shape=[
            jax.ShapeDtypeStruct((N, D), jnp.float32),
            jax.ShapeDtypeStruct((8, D), jnp.float32),
        ],
    )(*S_chunks, *z_chunks, dinv2d, b)


def _lrelu(x):
    return jnp.where(x >= 0, x, 0.01 * x)


def _bn_from_stats(t, st_ref, g_ref, bt_ref):
    mean = st_ref[0:1, :] * (1.0 / N)
    var = st_ref[1:2, :] * (1.0 / N) - mean * mean
    return (t - mean) * jax.lax.rsqrt(var + 1e-5) * g_ref[...] + bt_ref[...]


def _kB_body(t_ref, st_ref, g_ref, bt_ref, w_ref, dinv_ref, *z_refs):
    i = pl.program_id(0)
    y = _lrelu(_bn_from_stats(t_ref[...], st_ref, g_ref, bt_ref))
    dinv = dinv_ref[pl.ds(i * RB, RB), :]
    zz = jnp.dot(y, w_ref[...], preferred_element_type=jnp.float32) * dinv
    for k, zr in enumerate(z_refs):
        zr[...] = zz[:, k * 128:(k + 1) * 128]


def _bn_mm_scale(t, st, g, bt, W, dinv2d):
    D_in, D_out = W.shape
    nch = D_out // 128
    return pl.pallas_call(
        _kB_body,
        grid=(GRID,),
        in_specs=[
            pl.BlockSpec((RB, D_in), lambda i: (i, 0)),
            pl.BlockSpec((8, D_in), lambda i: (0, 0)),
            pl.BlockSpec((1, D_in), lambda i: (0, 0)),
            pl.BlockSpec((1, D_in), lambda i: (0, 0)),
            pl.BlockSpec((D_in, D_out), lambda i: (0, 0)),
            pl.BlockSpec((N, 1), lambda i: (0, 0)),
        ],
        out_specs=[pl.BlockSpec((RB, 128), lambda i: (i, 0))] * nch,
        out_shape=[jax.ShapeDtypeStruct((N, 128), jnp.float32)] * nch,
    )(t, st, g.reshape(1, -1), bt.reshape(1, -1), W, dinv2d)


def _kC_body(t_ref, st_ref, g_ref, bt_ref, w1_ref, b1_ref, w2_ref, b2_ref,
             o_ref):
    y = _lrelu(_bn_from_stats(t_ref[...], st_ref, g_ref, bt_ref))
    h = _lrelu(jnp.dot(y, w1_ref[...], preferred_element_type=jnp.float32)
               + b1_ref[...])
    logits = jnp.dot(h, w2_ref[...], preferred_element_type=jnp.float32) \
        + b2_ref[...]
    col = jax.lax.broadcasted_iota(jnp.int32, logits.shape, 1)
    logits = jnp.where(col < 6, logits, -1e30)
    m = jnp.max(logits, axis=1, keepdims=True)
    e = jnp.exp(logits - m)
    o_ref[...] = e / jnp.sum(e, axis=1, keepdims=True)


def _head(t, st, g, bt, Wd1, bd1, Wd2p, bd2p):
    D = t.shape[1]
    H = Wd1.shape[1]
    return pl.pallas_call(
        _kC_body,
        grid=(GRID,),
        in_specs=[
            pl.BlockSpec((RB, D), lambda i: (i, 0)),
            pl.BlockSpec((8, D), lambda i: (0, 0)),
            pl.BlockSpec((1, D), lambda i: (0, 0)),
            pl.BlockSpec((1, D), lambda i: (0, 0)),
            pl.BlockSpec((D, H), lambda i: (0, 0)),
            pl.BlockSpec((1, H), lambda i: (0, 0)),
            pl.BlockSpec((H, 128), lambda i: (0, 0)),
            pl.BlockSpec((1, 128), lambda i: (0, 0)),
        ],
        out_specs=pl.BlockSpec((RB, 128), lambda i: (i, 0)),
        out_shape=jax.ShapeDtypeStruct((N, 128), jnp.float32),
    )(t, st, g.reshape(1, -1), bt.reshape(1, -1), Wd1, bd1.reshape(1, -1),
      Wd2p, bd2p.reshape(1, -1))


HPAD = 10240
DEGR = EROWS // NS


@functools.lru_cache(maxsize=None)
def _make_degree():
    @functools.partial(
        pl.kernel,
        out_type=jax.ShapeDtypeStruct((HPAD, IDXW), jnp.float32),
        mesh=_mesh(),
        scratch_types=[
            pltpu.VMEM_SHARED((HPAD, IDXW), jnp.float32),
            pltpu.VMEM((16, IDXW), jnp.int32),
            pltpu.VMEM((IDXW, IDXW), jnp.float32),
            pltpu.VMEM((IDXW, IDXW), jnp.float32),
        ],
    )
    def degk(dst2d_hbm, deg_out, acc, idxv, onesb, zerob):
        c = lax.axis_index("c")
        s = lax.axis_index("s")

        def fill(i, _):
            for cc in range(IDXW // LANES):
                onesb[i, pl.ds(cc * LANES, LANES)] = jnp.ones((LANES,),
                                                              jnp.float32)
                zerob[i, pl.ds(cc * LANES, LANES)] = jnp.zeros((LANES,),
                                                               jnp.float32)
            return 0
        lax.fori_loop(0, IDXW, fill, 0)

        for q in range(HPAD // NS // IDXW):
            pltpu.sync_copy(zerob, acc.at[pl.ds(s * (HPAD // NS) + q * IDXW,
                                                IDXW)])
        plsc.subcore_barrier()

        @pl.when(c == 0)
        def _():
            def blk_body(blk, _):
                pltpu.sync_copy(
                    dst2d_hbm.at[pl.ds(s * DEGR + blk * 16, 16)], idxv)
                for j in range(16):
                    pltpu.sync_copy(onesb, acc.at[idxv.at[j]], add=True)
                return 0
            lax.fori_loop(0, DEGR // 16, blk_body, 0)
        plsc.subcore_barrier()

        @pl.when(c == 0)
        def _():
            for q in range(HPAD // NS // IDXW):
                pltpu.sync_copy(
                    acc.at[pl.ds(s * (HPAD // NS) + q * IDXW, IDXW)],
                    deg_out.at[pl.ds(s * (HPAD // NS) + q * IDXW, IDXW)])

    return degk


@functools.lru_cache(maxsize=None)
def _make_segsum(nchunks):
    npass = nchunks // NC
    QN = RPT // 8

    @functools.partial(
        pl.kernel,
        out_type=[jax.ShapeDtypeStruct((ACC_N, IDXW), jnp.float32)] * nchunks,
        mesh=_mesh(),
        scratch_types=[
            pltpu.VMEM_SHARED((ACC_N, IDXW), jnp.float32),
            pltpu.VMEM((3, 8, IDXW), jnp.int32),
            pltpu.VMEM((3, 8, IDXW), jnp.int32),
            pltpu.VMEM((IDXW, IDXW), jnp.float32),
            pltpu.VMEM((IDXW, IDXW), jnp.float32),
            pltpu.SemaphoreType.DMA,
            pltpu.SemaphoreType.DMA,
            pltpu.SemaphoreType.DMA,
            pltpu.SemaphoreType.DMA,
            pltpu.SemaphoreType.DMA,
            pltpu.SemaphoreType.DMA,
        ],
    )
    def seg(*args):
        zs = args[:nchunks]
        srcp, dstp = args[nchunks:nchunks + 2]
        outs = args[nchunks + 2:2 * nchunks + 2]
        scr = args[2 * nchunks + 2:]
        acc, srcb, dstb = scr[0:3]
        rowbufs = scr[3:5]
        gsems = scr[5:7]
        ssems = scr[7:9]
        isems = scr[9:11]
        c = lax.axis_index("c")
        s = lax.axis_index("s")

        for p in range(npass):
            def zr(i, _):
                for cc in range(IDXW // LANES):
                    rowbufs[0][i, pl.ds(cc * LANES, LANES)] = \
                        jnp.zeros((LANES,), jnp.float32)
                return 0
            lax.fori_loop(0, IDXW, zr, 0)
            for q in range(RPS // IDXW):
                pltpu.sync_copy(rowbufs[0],
                                acc.at[pl.ds(s * RPS + q * IDXW, IDXW)])
            pltpu.sync_copy(srcp.at[s * QN], srcb.at[0])
            pltpu.sync_copy(dstp.at[s * QN], dstb.at[0])
            plsc.subcore_barrier()

            for cid in range(NC):
                @pl.when(c == cid)
                def _(cid=cid):
                    z_ref = zs[p * NC + cid]

                    def wait_scatter(slot):
                        pltpu.make_async_copy(rowbufs[slot],
                                              acc.at[dstb.at[0, 0]],
                                              ssems[slot]).wait()

                    def retire(prev, par, bd):
                        pltpu.make_async_copy(z_ref.at[srcb.at[0, 0]],
                                              rowbufs[prev],
                                              gsems[prev]).wait()
                        pltpu.async_copy(rowbufs[prev],
                                         acc.at[dstb.at[par, bd]],
                                         ssems[prev], add=True)

                    def qbody(q, _):
                        par = lax.rem(q, 3)
                        parp = lax.rem(q + 2, 3)
                        parn = lax.rem(q + 1, 3)

                        @pl.when(q > 0)
                        def _():
                            pltpu.make_async_copy(srcp.at[s * QN],
                                                  srcb.at[0],
                                                  isems[0]).wait()
                            pltpu.make_async_copy(dstp.at[s * QN],
                                                  dstb.at[0],
                                                  isems[1]).wait()

                        @pl.when(q < QN - 1)
                        def _():
                            pltpu.async_copy(srcp.at[s * QN + q + 1],
                                             srcb.at[parn], isems[0])
                            pltpu.async_copy(dstp.at[s * QN + q + 1],
                                             dstb.at[parn], isems[1])

                        for bb in range(8):
                            slot = bb % 2
                            prev = (bb - 1) % 2
                            if bb < 2:
                                @pl.when(q > 0)
                                def _(slot=slot):
                                    wait_scatter(slot)
                            else:
                                wait_scatter(slot)
                            pltpu.async_copy(z_ref.at[srcb.at[par, bb]],
                                             rowbufs[slot], gsems[slot])
                            if bb == 0:
                                @pl.when(q > 0)
                                def _(prev=prev, parp=parp):
                                    retire(prev, parp, 7)
                            else:
                                retire(prev, par, bb - 1)
                        return 0
                    lax.fori_loop(0, QN, qbody, 0)
                    retire(1, (QN - 1) % 3, 7)
                    wait_scatter(0)
                    wait_scatter(1)
            plsc.subcore_barrier()

            for cid in range(NC):
                @pl.when(c == cid)
                def _(cid=cid):
                    o_ref = outs[p * NC + cid]
                    for q in range(RPS // IDXW):
                        pltpu.sync_copy(
                            acc.at[pl.ds(s * RPS + q * IDXW, IDXW)],
                            o_ref.at[pl.ds(s * RPS + q * IDXW, IDXW)])
            if p + 1 < npass:
                plsc.subcore_barrier()

    return seg


def _edge_rows(edge_index):
    pad = EROWS * IDXW - E
    srcp = jnp.concatenate(
        [edge_index[0], jnp.zeros((pad,), jnp.int32)]).reshape(EROWS, IDXW)
    dstp = jnp.concatenate(
        [edge_index[1],
         jnp.full((pad,), DUMP, jnp.int32)]).reshape(EROWS, IDXW)
    return srcp.reshape(EROWS // 8, 8, IDXW), dstp.reshape(EROWS // 8, 8, IDXW)


def _deg(dstp):
    return _make_degree()(dstp.reshape(EROWS, IDXW))[:N, 0:1]


def _segsum(z_chunks, srcp, dstp):
    return _make_segsum(len(z_chunks))(*z_chunks, srcp, dstp)


def kernel(x, edge_index, W1, b1, g1, bt1, W2, b2, g2, bt2, Wd1, bd1, Wd2,
           bd2):
    srcp, dstp = _edge_rows(edge_index)
    deg2d = _deg(dstp)
    z1c, dinv2d = _mm_scale(x, W1, deg2d)
    S1c = _segsum(z1c, srcp, dstp)
    t1, st1 = _post_conv(S1c, z1c, dinv2d, b1.reshape(1, -1))
    z2c = _bn_mm_scale(t1, st1, g1, bt1, W2, dinv2d)
    S2c = _segsum(z2c, srcp, dstp)
    t2, st2 = _post_conv(S2c, z2c, dinv2d, b2.reshape(1, -1))
    Wd2p = jnp.pad(Wd2, ((0, 0), (0, 128 - Wd2.shape[1])))
    bd2p = jnp.pad(bd2, (0, 128 - bd2.shape[0]))
    out = _head(t2, st2, g2, bt2, Wd1, bd1, Wd2p, bd2p)
    return out[:, :6]


# speedup vs baseline: 7.5624x; 1.0606x over previous
import functools
import jax
import jax.numpy as jnp
from jax import lax
from jax.experimental import pallas as pl
from jax.experimental.pallas import tpu as pltpu
from jax.experimental.pallas import tpu_sc as plsc

N = 10000
RB = 400
GRID = N // RB

NC, NS, LANES = 2, 16, 16
E = 320000
IDXW = 128
EROWS = 2560
RPT = EROWS // NS
STG = 8
ACC_N = 10240
DUMP = 10100
RPS = ACC_N // NS


@functools.lru_cache(maxsize=None)
def _mesh():
    return plsc.VectorSubcoreMesh(core_axis_name="c", subcore_axis_name="s",
                                  num_cores=NC, num_subcores=NS)


def _k1_body(x_ref, w_ref, deg_ref, *out_refs):
    i = pl.program_id(0)
    z_refs, dinv_ref = out_refs[:-1], out_refs[-1]
    dinv = jax.lax.rsqrt(deg_ref[pl.ds(i * RB, RB), :] + 1.0)
    zz = jnp.dot(x_ref[...], w_ref[...],
                 preferred_element_type=jnp.float32) * dinv
    for k, zr in enumerate(z_refs):
        zr[...] = zz[:, k * 128:(k + 1) * 128]
    dinv_ref[...] = jax.lax.rsqrt(deg_ref[...] + 1.0)


def _mm_scale(x, W, deg2d):
    D_in, D_out = W.shape
    nch = D_out // 128
    out = pl.pallas_call(
        _k1_body,
        grid=(GRID,),
        in_specs=[
            pl.BlockSpec((RB, D_in), lambda i: (i, 0)),
            pl.BlockSpec((D_in, D_out), lambda i: (0, 0)),
            pl.BlockSpec((N, 1), lambda i: (0, 0)),
        ],
        out_specs=[pl.BlockSpec((RB, 128), lambda i: (i, 0))] * nch
        + [pl.BlockSpec((N, 1), lambda i: (0, 0))],
        out_shape=[jax.ShapeDtypeStruct((N, 128), jnp.float32)] * nch
        + [jax.ShapeDtypeStruct((N, 1), jnp.float32)],
    )(x, W, deg2d)
    return out[:-1], out[-1]


def _kA_body(*refs):
    nch = (len(refs) - 4) // 2
    s_refs = refs[:nch]
    z_refs = refs[nch:2 * nch]
    dinv_ref, b_ref, t_ref, st_ref = refs[2 * nch:]
    i = pl.program_id(0)
    dinv = dinv_ref[pl.ds(i * RB, RB), :]
    S = jnp.concatenate([r[...] for r in s_refs], axis=1)
    z = jnp.concatenate([r[...] for r in z_refs], axis=1)
    t = dinv * (S + z) + b_ref[...]
    t_ref[...] = t
    colsum = jnp.sum(t, axis=0, keepdims=True)
    colsq = jnp.sum(t * t, axis=0, keepdims=True)
    upd = jnp.concatenate([colsum, colsq, jnp.zeros((6, t.shape[1]),
                                                    jnp.float32)], axis=0)

    @pl.when(i == 0)
    def _():
        st_ref[...] = jnp.zeros_like(st_ref)

    st_ref[...] += upd


def _post_conv(S_chunks, z_chunks, dinv2d, b):
    nch = len(z_chunks)
    D = nch * 128
    return pl.pallas_call(
        _kA_body,
        grid=(GRID,),
        in_specs=[pl.BlockSpec((RB, 128), lambda i: (i, 0))] * (2 * nch)
        + [
            pl.BlockSpec((N, 1), lambda i: (0, 0)),
            pl.BlockSpec((1, D), lambda i: (0, 0)),
        ],
        out_specs=[
            pl.BlockSpec((RB, D), lambda i: (i, 0)),
            pl.BlockSpec((8, D), lambda i: (0, 0)),
        ],
        out_shape=[
            jax.ShapeDtypeStruct((N, D), jnp.float32),
            jax.ShapeDtypeStruct((8, D), jnp.float32),
        ],
    )(*S_chunks, *z_chunks, dinv2d, b)


def _lrelu(x):
    return jnp.where(x >= 0, x, 0.01 * x)


def _bn_from_stats(t, st_ref, g_ref, bt_ref):
    mean = st_ref[0:1, :] * (1.0 / N)
    var = st_ref[1:2, :] * (1.0 / N) - mean * mean
    return (t - mean) * jax.lax.rsqrt(var + 1e-5) * g_ref[...] + bt_ref[...]


def _kB_body(t_ref, st_ref, g_ref, bt_ref, w_ref, dinv_ref, *z_refs):
    i = pl.program_id(0)
    y = _lrelu(_bn_from_stats(t_ref[...], st_ref, g_ref, bt_ref))
    dinv = dinv_ref[pl.ds(i * RB, RB), :]
    zz = jnp.dot(y, w_ref[...], preferred_element_type=jnp.float32) * dinv
    for k, zr in enumerate(z_refs):
        zr[...] = zz[:, k * 128:(k + 1) * 128]


def _bn_mm_scale(t, st, g, bt, W, dinv2d):
    D_in, D_out = W.shape
    nch = D_out // 128
    return pl.pallas_call(
        _kB_body,
        grid=(GRID,),
        in_specs=[
            pl.BlockSpec((RB, D_in), lambda i: (i, 0)),
            pl.BlockSpec((8, D_in), lambda i: (0, 0)),
            pl.BlockSpec((1, D_in), lambda i: (0, 0)),
            pl.BlockSpec((1, D_in), lambda i: (0, 0)),
            pl.BlockSpec((D_in, D_out), lambda i: (0, 0)),
            pl.BlockSpec((N, 1), lambda i: (0, 0)),
        ],
        out_specs=[pl.BlockSpec((RB, 128), lambda i: (i, 0))] * nch,
        out_shape=[jax.ShapeDtypeStruct((N, 128), jnp.float32)] * nch,
    )(t, st, g.reshape(1, -1), bt.reshape(1, -1), W, dinv2d)


def _kC_body(t_ref, st_ref, g_ref, bt_ref, w1_ref, b1_ref, w2_ref, b2_ref,
             o_ref):
    y = _lrelu(_bn_from_stats(t_ref[...], st_ref, g_ref, bt_ref))
    h = _lrelu(jnp.dot(y.astype(jnp.bfloat16),
                       w1_ref[...].astype(jnp.bfloat16),
                       preferred_element_type=jnp.float32) + b1_ref[...])
    logits = jnp.dot(h.astype(jnp.bfloat16),
                     w2_ref[...].astype(jnp.bfloat16),
                     preferred_element_type=jnp.float32) + b2_ref[...]
    col = jax.lax.broadcasted_iota(jnp.int32, logits.shape, 1)
    logits = jnp.where(col < 6, logits, -1e30)
    m = jnp.max(logits, axis=1, keepdims=True)
    e = jnp.exp(logits - m)
    o_ref[...] = e / jnp.sum(e, axis=1, keepdims=True)


def _head(t, st, g, bt, Wd1, bd1, Wd2p, bd2p):
    D = t.shape[1]
    H = Wd1.shape[1]
    return pl.pallas_call(
        _kC_body,
        grid=(GRID,),
        in_specs=[
            pl.BlockSpec((RB, D), lambda i: (i, 0)),
            pl.BlockSpec((8, D), lambda i: (0, 0)),
            pl.BlockSpec((1, D), lambda i: (0, 0)),
            pl.BlockSpec((1, D), lambda i: (0, 0)),
            pl.BlockSpec((D, H), lambda i: (0, 0)),
            pl.BlockSpec((1, H), lambda i: (0, 0)),
            pl.BlockSpec((H, 128), lambda i: (0, 0)),
            pl.BlockSpec((1, 128), lambda i: (0, 0)),
        ],
        out_specs=pl.BlockSpec((RB, 128), lambda i: (i, 0)),
        out_shape=jax.ShapeDtypeStruct((N, 128), jnp.float32),
    )(t, st, g.reshape(1, -1), bt.reshape(1, -1), Wd1, bd1.reshape(1, -1),
      Wd2p, bd2p.reshape(1, -1))


HPAD = 10240
DEGR = EROWS // NS


@functools.lru_cache(maxsize=None)
def _make_degree():
    QN = DEGR // 16

    @functools.partial(
        pl.kernel,
        out_type=jax.ShapeDtypeStruct((HPAD, IDXW), jnp.float32),
        mesh=_mesh(),
        scratch_types=[
            pltpu.VMEM_SHARED((HPAD, IDXW), jnp.float32),
            pltpu.VMEM((2, 16, IDXW), jnp.int32),
            pltpu.VMEM((IDXW, IDXW), jnp.float32),
            pltpu.VMEM((IDXW, IDXW), jnp.float32),
            pltpu.SemaphoreType.DMA,
            pltpu.SemaphoreType.DMA,
        ],
    )
    def degk(dst3d_hbm, deg_out, acc, idxv, onesb, zerob, ssem, isem):
        c = lax.axis_index("c")
        s = lax.axis_index("s")

        def fill(i, _):
            for cc in range(IDXW // LANES):
                onesb[i, pl.ds(cc * LANES, LANES)] = jnp.ones((LANES,),
                                                              jnp.float32)
                zerob[i, pl.ds(cc * LANES, LANES)] = jnp.zeros((LANES,),
                                                               jnp.float32)
            return 0
        lax.fori_loop(0, IDXW, fill, 0)

        for q in range(HPAD // NS // IDXW):
            pltpu.sync_copy(zerob, acc.at[pl.ds(s * (HPAD // NS) + q * IDXW,
                                                IDXW)])
        plsc.subcore_barrier()

        @pl.when(c == 0)
        def _():
            pltpu.sync_copy(dst3d_hbm.at[s * QN], idxv.at[0])

            def qbody(q, _):
                par = lax.rem(q, 2)
                parn = lax.rem(q + 1, 2)

                @pl.when(q > 0)
                def _():
                    pltpu.make_async_copy(dst3d_hbm.at[s * QN],
                                          idxv.at[0], isem).wait()

                @pl.when(q < QN - 1)
                def _():
                    pltpu.async_copy(dst3d_hbm.at[s * QN + q + 1],
                                     idxv.at[parn], isem)

                for j in range(16):
                    pltpu.async_copy(onesb, acc.at[idxv.at[par, j]],
                                     ssem, add=True)
                for j in range(16):
                    pltpu.make_async_copy(onesb, acc.at[idxv.at[par, 0]],
                                          ssem).wait()
                return 0
            lax.fori_loop(0, QN, qbody, 0)
        plsc.subcore_barrier()

        @pl.when(c == 0)
        def _():
            for q in range(HPAD // NS // IDXW):
                pltpu.sync_copy(
                    acc.at[pl.ds(s * (HPAD // NS) + q * IDXW, IDXW)],
                    deg_out.at[pl.ds(s * (HPAD // NS) + q * IDXW, IDXW)])

    return degk


@functools.lru_cache(maxsize=None)
def _make_segsum(nchunks):
    npass = nchunks // NC
    QN = RPT // 8

    @functools.partial(
        pl.kernel,
        out_type=[jax.ShapeDtypeStruct((ACC_N, IDXW), jnp.float32)] * nchunks,
        mesh=_mesh(),
        scratch_types=[
            pltpu.VMEM_SHARED((ACC_N, IDXW), jnp.float32),
            pltpu.VMEM((3, 8, IDXW), jnp.int32),
            pltpu.VMEM((3, 8, IDXW), jnp.int32),
            pltpu.VMEM((IDXW, IDXW), jnp.float32),
            pltpu.VMEM((IDXW, IDXW), jnp.float32),
            pltpu.SemaphoreType.DMA,
            pltpu.SemaphoreType.DMA,
            pltpu.SemaphoreType.DMA,
            pltpu.SemaphoreType.DMA,
            pltpu.SemaphoreType.DMA,
            pltpu.SemaphoreType.DMA,
        ],
    )
    def seg(*args):
        zs = args[:nchunks]
        srcp, dstp = args[nchunks:nchunks + 2]
        outs = args[nchunks + 2:2 * nchunks + 2]
        scr = args[2 * nchunks + 2:]
        acc, srcb, dstb = scr[0:3]
        rowbufs = scr[3:5]
        gsems = scr[5:7]
        ssems = scr[7:9]
        isems = scr[9:11]
        c = lax.axis_index("c")
        s = lax.axis_index("s")

        for p in range(npass):
            def zr(i, _):
                for cc in range(IDXW // LANES):
                    rowbufs[0][i, pl.ds(cc * LANES, LANES)] = \
                        jnp.zeros((LANES,), jnp.float32)
                return 0
            lax.fori_loop(0, IDXW, zr, 0)
            for q in range(RPS // IDXW):
                pltpu.sync_copy(rowbufs[0],
                                acc.at[pl.ds(s * RPS + q * IDXW, IDXW)])
            pltpu.sync_copy(srcp.at[s * QN], srcb.at[0])
            pltpu.sync_copy(dstp.at[s * QN], dstb.at[0])
            plsc.subcore_barrier()

            for cid in range(NC):
                @pl.when(c == cid)
                def _(cid=cid):
                    z_ref = zs[p * NC + cid]

                    def wait_scatter(slot):
                        pltpu.make_async_copy(rowbufs[slot],
                                              acc.at[dstb.at[0, 0]],
                                              ssems[slot]).wait()

                    def retire(prev, par, bd):
                        pltpu.make_async_copy(z_ref.at[srcb.at[0, 0]],
                                              rowbufs[prev],
                                              gsems[prev]).wait()
                        pltpu.async_copy(rowbufs[prev],
                                         acc.at[dstb.at[par, bd]],
                                         ssems[prev], add=True)

                    def qbody(q, _):
                        par = lax.rem(q, 3)
                        parp = lax.rem(q + 2, 3)
                        parn = lax.rem(q + 1, 3)

                        @pl.when(q > 0)
                        def _():
                            pltpu.make_async_copy(srcp.at[s * QN],
                                                  srcb.at[0],
                                                  isems[0]).wait()
                            pltpu.make_async_copy(dstp.at[s * QN],
                                                  dstb.at[0],
                                                  isems[1]).wait()

                        @pl.when(q < QN - 1)
                        def _():
                            pltpu.async_copy(srcp.at[s * QN + q + 1],
                                             srcb.at[parn], isems[0])
                            pltpu.async_copy(dstp.at[s * QN + q + 1],
                                             dstb.at[parn], isems[1])

                        for bb in range(8):
                            slot = bb % 2
                            prev = (bb - 1) % 2
                            if bb < 2:
                                @pl.when(q > 0)
                                def _(slot=slot):
                                    wait_scatter(slot)
                            else:
                                wait_scatter(slot)
                            pltpu.async_copy(z_ref.at[srcb.at[par, bb]],
                                             rowbufs[slot], gsems[slot])
                            if bb == 0:
                                @pl.when(q > 0)
                                def _(prev=prev, parp=parp):
                                    retire(prev, parp, 7)
                            else:
                                retire(prev, par, bb - 1)
                        return 0
                    lax.fori_loop(0, QN, qbody, 0)
                    retire(1, (QN - 1) % 3, 7)
                    wait_scatter(0)
                    wait_scatter(1)
            plsc.subcore_barrier()

            for cid in range(NC):
                @pl.when(c == cid)
                def _(cid=cid):
                    o_ref = outs[p * NC + cid]
                    for q in range(RPS // IDXW):
                        pltpu.sync_copy(
                            acc.at[pl.ds(s * RPS + q * IDXW, IDXW)],
                            o_ref.at[pl.ds(s * RPS + q * IDXW, IDXW)])
            if p + 1 < npass:
                plsc.subcore_barrier()

    return seg


def _edge_rows(edge_index):
    pad = EROWS * IDXW - E
    srcp = jnp.concatenate(
        [edge_index[0], jnp.zeros((pad,), jnp.int32)]).reshape(EROWS, IDXW)
    dstp = jnp.concatenate(
        [edge_index[1],
         jnp.full((pad,), DUMP, jnp.int32)]).reshape(EROWS, IDXW)
    return srcp.reshape(EROWS // 8, 8, IDXW), dstp.reshape(EROWS // 8, 8, IDXW)


def _deg(dstp):
    return _make_degree()(
        dstp.reshape(EROWS // 16, 16, IDXW))[:N, 0:1]


def _segsum(z_chunks, srcp, dstp):
    return _make_segsum(len(z_chunks))(*z_chunks, srcp, dstp)


def kernel(x, edge_index, W1, b1, g1, bt1, W2, b2, g2, bt2, Wd1, bd1, Wd2,
           bd2):
    srcp, dstp = _edge_rows(edge_index)
    deg2d = _deg(dstp)
    z1c, dinv2d = _mm_scale(x, W1, deg2d)
    S1c = _segsum(z1c, srcp, dstp)
    t1, st1 = _post_conv(S1c, z1c, dinv2d, b1.reshape(1, -1))
    z2c = _bn_mm_scale(t1, st1, g1, bt1, W2, dinv2d)
    S2c = _segsum(z2c, srcp, dstp)
    t2, st2 = _post_conv(S2c, z2c, dinv2d, b2.reshape(1, -1))
    Wd2p = jnp.pad(Wd2, ((0, 0), (0, 128 - Wd2.shape[1])))
    bd2p = jnp.pad(bd2, (0, 128 - bd2.shape[0]))
    out = _head(t2, st2, g2, bt2, Wd1, bd1, Wd2p, bd2p)
    return out[:, :6]
